# baseline (speedup 1.0000x reference)
import jax
import jax.numpy as jnp
from jax import lax
from jax.experimental import pallas as pl
from jax.experimental.pallas import tpu as pltpu
import jax.experimental.pallas.tpu_sc as plsc

B, E, A = 4096, 8, 1000
L = 16
NC, NS = 2, 16
NW = NC * NS
ROWS_PER_W = B // NW
NCHUNK = (A + L - 1) // L
A_PAD = NCHUNK * L


def _sc_body(a_hbm, er_hbm, o_hbm, a_v, e_v, s_v):
    wid = lax.axis_index("s") * NC + lax.axis_index("c")
    base = wid * ROWS_PER_W
    lane = lax.iota(jnp.int32, L)

    def row_body(i, carry):
        row = base + i
        pltpu.sync_copy(a_hbm.at[row], a_v.at[:, pl.ds(0, A)])
        pltpu.sync_copy(er_hbm.at[row], e_v.at[:, pl.ds(0, A)])

        def p1(cidx, m):
            off = pl.multiple_of(cidx * L, L)
            best = a_v[0, pl.ds(off, L)]
            bstd = e_v[0, pl.ds(off, L)]
            for e in range(1, E):
                ak = a_v[e, pl.ds(off, L)]
                ek = e_v[e, pl.ds(off, L)]
                take = ak < best
                bstd = jnp.where(take, ek, bstd)
                best = jnp.where(take, ak, best)
            scaled = best / jnp.maximum(bstd, 1e-8)
            scaled = jnp.where(lane + off < A, scaled, -3e38)
            s_v[pl.ds(off, L)] = scaled
            return jnp.maximum(m, scaled)

        m = lax.fori_loop(0, NCHUNK, p1, jnp.full((L,), -3e38, jnp.float32))
        row_max = jnp.max(m)

        def p2(cidx, acc):
            off = pl.multiple_of(cidx * L, L)
            v = s_v[pl.ds(off, L)]
            p = jnp.exp(v - row_max)
            p = jnp.where(lane + off < A, p, 0.0)
            s_v[pl.ds(off, L)] = p
            return acc + p

        acc = lax.fori_loop(0, NCHUNK, p2, jnp.zeros((L,), jnp.float32))
        inv = 1.0 / (jnp.sum(acc) + jnp.zeros((L,), jnp.float32))

        def p3(cidx, c):
            off = pl.multiple_of(cidx * L, L)
            s_v[pl.ds(off, L)] = s_v[pl.ds(off, L)] * inv
            return c

        lax.fori_loop(0, NCHUNK, p3, 0)
        pltpu.sync_copy(s_v.at[pl.ds(0, A)], o_hbm.at[row])
        return carry

    lax.fori_loop(0, ROWS_PER_W, row_body, 0)


@jax.jit
def _sc_call(action, explore_rate):
    return pl.kernel(
        _sc_body,
        out_type=jax.ShapeDtypeStruct((B, A), jnp.float32),
        mesh=plsc.VectorSubcoreMesh(
            core_axis_name="c", subcore_axis_name="s",
            num_cores=NC, num_subcores=NS,
        ),
        scratch_types=[
            pltpu.VMEM((E, A_PAD), jnp.float32),
            pltpu.VMEM((E, A_PAD), jnp.float32),
            pltpu.VMEM((A_PAD,), jnp.float32),
        ],
        compiler_params=pltpu.CompilerParams(
            use_tc_tiling_on_sc=False, needs_layout_passes=False,
        ),
    )(action, explore_rate)


def kernel(action, explore_rate, step, obs):
    del step, obs
    return _sc_call(action, explore_rate)


# speedup vs baseline: 1.5168x; 1.5168x over previous
import jax
import jax.numpy as jnp
from jax import lax
from jax.experimental import pallas as pl
from jax.experimental.pallas import tpu as pltpu
import jax.experimental.pallas.tpu_sc as plsc

B, E, A = 4096, 8, 1000
L = 16
NC, NS = 2, 16
NW = NC * NS
ROWS_PER_W = B // NW
NCHUNK = (A + L - 1) // L
A_PAD = NCHUNK * L
UNROLL = 3
NBUF = 2


def _sc_body(a_hbm, er_hbm, o_hbm, a_v, e_v, s_v, in_sem, out_sem):
    wid = lax.axis_index("s") * NC + lax.axis_index("c")
    base = wid * ROWS_PER_W
    lane = lax.iota(jnp.int32, L)

    def start_in(j, row):
        pltpu.async_copy(a_hbm.at[row], a_v.at[j, :, pl.ds(0, A)], in_sem.at[j])
        pltpu.async_copy(er_hbm.at[row], e_v.at[j, :, pl.ds(0, A)], in_sem.at[j])

    def wait_in(j, row):
        pltpu.make_async_copy(a_hbm.at[row], a_v.at[j, :, pl.ds(0, A)], in_sem.at[j]).wait()
        pltpu.make_async_copy(er_hbm.at[row], e_v.at[j, :, pl.ds(0, A)], in_sem.at[j]).wait()

    def compute(j, row):
        def p1(it, m):
            for u in range(UNROLL):
                off = pl.multiple_of(it * (L * UNROLL), L) + u * L
                best = a_v[j, 0, pl.ds(off, L)]
                bstd = e_v[j, 0, pl.ds(off, L)]
                for e in range(1, E):
                    ak = a_v[j, e, pl.ds(off, L)]
                    ek = e_v[j, e, pl.ds(off, L)]
                    take = ak < best
                    bstd = jnp.where(take, ek, bstd)
                    best = jnp.where(take, ak, best)
                scaled = best / jnp.maximum(bstd, 1e-8)
                scaled = jnp.where(lane + off < A, scaled, -3e38)
                s_v[j, pl.ds(off, L)] = scaled
                m = jnp.maximum(m, scaled)
            return m

        m = lax.fori_loop(0, NCHUNK // UNROLL, p1,
                          jnp.full((L,), -3e38, jnp.float32))
        row_max = jnp.max(m)

        def p2(it, acc):
            for u in range(UNROLL):
                off = pl.multiple_of(it * (L * UNROLL), L) + u * L
                v = s_v[j, pl.ds(off, L)]
                p = jnp.exp(v - row_max)
                p = jnp.where(lane + off < A, p, 0.0)
                s_v[j, pl.ds(off, L)] = p
                acc = acc + p
            return acc

        acc = lax.fori_loop(0, NCHUNK // UNROLL, p2, jnp.zeros((L,), jnp.float32))
        inv = 1.0 / (jnp.sum(acc) + jnp.zeros((L,), jnp.float32))

        def p3(it, c):
            for u in range(UNROLL):
                off = pl.multiple_of(it * (L * UNROLL), L) + u * L
                s_v[j, pl.ds(off, L)] = s_v[j, pl.ds(off, L)] * inv
            return c

        lax.fori_loop(0, NCHUNK // UNROLL, p3, 0)

    for j in range(NBUF):
        start_in(j, base + j)

    def blk(g, carry):
        for j in range(NBUF):
            i = g * NBUF + j
            row = base + i
            wait_in(j, row)

            @pl.when(g > 0)
            def _():
                pltpu.make_async_copy(
                    s_v.at[j, pl.ds(0, A)], o_hbm.at[row - NBUF], out_sem.at[j]
                ).wait()

            compute(j, row)
            pltpu.async_copy(s_v.at[j, pl.ds(0, A)], o_hbm.at[row], out_sem.at[j])

            @pl.when(g < ROWS_PER_W // NBUF - 1)
            def _():
                start_in(j, row + NBUF)
        return carry

    lax.fori_loop(0, ROWS_PER_W // NBUF, blk, 0)

    for j in range(NBUF):
        row = base + ROWS_PER_W - NBUF + j
        pltpu.make_async_copy(
            s_v.at[j, pl.ds(0, A)], o_hbm.at[row], out_sem.at[j]
        ).wait()


@jax.jit
def _sc_call(action, explore_rate):
    return pl.kernel(
        _sc_body,
        out_type=jax.ShapeDtypeStruct((B, A), jnp.float32),
        mesh=plsc.VectorSubcoreMesh(
            core_axis_name="c", subcore_axis_name="s",
            num_cores=NC, num_subcores=NS,
        ),
        scratch_types=[
            pltpu.VMEM((NBUF, E, A_PAD), jnp.float32),
            pltpu.VMEM((NBUF, E, A_PAD), jnp.float32),
            pltpu.VMEM((NBUF, A_PAD), jnp.float32),
            pltpu.SemaphoreType.DMA((NBUF,)),
            pltpu.SemaphoreType.DMA((NBUF,)),
        ],
        compiler_params=pltpu.CompilerParams(
            use_tc_tiling_on_sc=False, needs_layout_passes=False,
        ),
    )(action, explore_rate)


def kernel(action, explore_rate, step, obs):
    del step, obs
    return _sc_call(action, explore_rate)


# speedup vs baseline: 1.6488x; 1.0870x over previous
import jax
import jax.numpy as jnp
from jax import lax
from jax.experimental import pallas as pl
from jax.experimental.pallas import tpu as pltpu
import jax.experimental.pallas.tpu_sc as plsc

B, E, A = 4096, 8, 1000
L = 16
NC, NS = 2, 16
NW = NC * NS
ROWS_PER_W = B // NW
NCHUNK = (A + L - 1) // L
A_PAD = NCHUNK * L
UNROLL = 3
NBUF = 2


def _sc_body(a_hbm, er_hbm, o_hbm, a_v, e_v, s_v, in_sem, out_sem):
    wid = lax.axis_index("s") * NC + lax.axis_index("c")
    base = wid * ROWS_PER_W
    lane = lax.iota(jnp.int32, L)

    def start_in(j, row):
        pltpu.async_copy(a_hbm.at[row], a_v.at[j, :, pl.ds(0, A)], in_sem.at[j])
        pltpu.async_copy(er_hbm.at[row], e_v.at[j, :, pl.ds(0, A)], in_sem.at[j])

    def wait_in(j, row):
        pltpu.make_async_copy(a_hbm.at[row], a_v.at[j, :, pl.ds(0, A)], in_sem.at[j]).wait()
        pltpu.make_async_copy(er_hbm.at[row], e_v.at[j, :, pl.ds(0, A)], in_sem.at[j]).wait()

    def compute(j, row):
        def p1(off, m):
            off = pl.multiple_of(off, L)
            best = a_v[j, 0, pl.ds(off, L)]
            bstd = e_v[j, 0, pl.ds(off, L)]
            for e in range(1, E):
                ak = a_v[j, e, pl.ds(off, L)]
                ek = e_v[j, e, pl.ds(off, L)]
                take = ak < best
                bstd = jnp.where(take, ek, bstd)
                best = jnp.where(take, ak, best)
            scaled = best / jnp.maximum(bstd, 1e-8)
            scaled = jnp.where(lane + off < A, scaled, -3e38)
            s_v[j, pl.ds(off, L)] = scaled
            return jnp.maximum(m, scaled)

        m = plsc.parallel_loop(
            0, A_PAD, L, unroll=UNROLL,
            carry=jnp.full((L,), -3e38, jnp.float32))(p1)
        row_max = jnp.max(m)

        def p2(off, acc):
            off = pl.multiple_of(off, L)
            v = s_v[j, pl.ds(off, L)]
            p = jnp.exp(v - row_max)
            p = jnp.where(lane + off < A, p, 0.0)
            s_v[j, pl.ds(off, L)] = p
            return acc + p

        acc = plsc.parallel_loop(
            0, A_PAD, L, unroll=UNROLL,
            carry=jnp.zeros((L,), jnp.float32))(p2)
        inv = 1.0 / (jnp.sum(acc) + jnp.zeros((L,), jnp.float32))

        def p3(off):
            off = pl.multiple_of(off, L)
            s_v[j, pl.ds(off, L)] = s_v[j, pl.ds(off, L)] * inv

        plsc.parallel_loop(0, A_PAD, L, unroll=UNROLL)(p3)

    for j in range(NBUF):
        start_in(j, base + j)

    def blk(g, carry):
        for j in range(NBUF):
            i = g * NBUF + j
            row = base + i
            wait_in(j, row)

            @pl.when(g > 0)
            def _():
                pltpu.make_async_copy(
                    s_v.at[j, pl.ds(0, A)], o_hbm.at[row - NBUF], out_sem.at[j]
                ).wait()

            compute(j, row)
            pltpu.async_copy(s_v.at[j, pl.ds(0, A)], o_hbm.at[row], out_sem.at[j])

            @pl.when(g < ROWS_PER_W // NBUF - 1)
            def _():
                start_in(j, row + NBUF)
        return carry

    lax.fori_loop(0, ROWS_PER_W // NBUF, blk, 0)

    for j in range(NBUF):
        row = base + ROWS_PER_W - NBUF + j
        pltpu.make_async_copy(
            s_v.at[j, pl.ds(0, A)], o_hbm.at[row], out_sem.at[j]
        ).wait()


@jax.jit
def _sc_call(action, explore_rate):
    return pl.kernel(
        _sc_body,
        out_type=jax.ShapeDtypeStruct((B, A), jnp.float32),
        mesh=plsc.VectorSubcoreMesh(
            core_axis_name="c", subcore_axis_name="s",
            num_cores=NC, num_subcores=NS,
        ),
        scratch_types=[
            pltpu.VMEM((NBUF, E, A_PAD), jnp.float32),
            pltpu.VMEM((NBUF, E, A_PAD), jnp.float32),
            pltpu.VMEM((NBUF, A_PAD), jnp.float32),
            pltpu.SemaphoreType.DMA((NBUF,)),
            pltpu.SemaphoreType.DMA((NBUF,)),
        ],
        compiler_params=pltpu.CompilerParams(
            use_tc_tiling_on_sc=False, needs_layout_passes=False,
        ),
    )(action, explore_rate)


def kernel(action, explore_rate, step, obs):
    del step, obs
    return _sc_call(action, explore_rate)


# speedup vs baseline: 2.5578x; 1.5513x over previous
import jax
import jax.numpy as jnp
from jax import lax
from jax.experimental import pallas as pl
from jax.experimental.pallas import tpu as pltpu
import jax.experimental.pallas.tpu_sc as plsc

B, E, A = 4096, 8, 1000
L = 16
NC, NS = 2, 16
NW = NC * NS
ROWS_PER_W = B // NW
NCHUNK = (A + L - 1) // L
A_PAD = NCHUNK * L
UNROLL = 3
NBUF = 2


def _sc_body(a_hbm, er_hbm, o_hbm, a_v, e_v, s_v, in_sem, out_sem):
    wid = lax.axis_index("s") * NC + lax.axis_index("c")
    base = wid * ROWS_PER_W
    lane = lax.iota(jnp.int32, L)

    def start_in(j, row):
        pltpu.async_copy(a_hbm.at[row], a_v.at[j], in_sem.at[j])
        pltpu.async_copy(er_hbm.at[row], e_v.at[j], in_sem.at[j])

    def wait_in(j, row):
        pltpu.make_async_copy(a_hbm.at[row], a_v.at[j], in_sem.at[j]).wait()
        pltpu.make_async_copy(er_hbm.at[row], e_v.at[j], in_sem.at[j]).wait()

    def compute(j, row):
        def p1(off, m):
            off = pl.multiple_of(off, L)
            best = a_v[j, 0, pl.ds(off, L)]
            bstd = e_v[j, 0, pl.ds(off, L)]
            for e in range(1, E):
                ak = a_v[j, e, pl.ds(off, L)]
                ek = e_v[j, e, pl.ds(off, L)]
                take = ak < best
                bstd = jnp.where(take, ek, bstd)
                best = jnp.where(take, ak, best)
            scaled = best / jnp.maximum(bstd, 1e-8)
            scaled = jnp.where(lane + off < A, scaled, -3e38)
            s_v[j, pl.ds(off, L)] = scaled
            return jnp.maximum(m, scaled)

        m = plsc.parallel_loop(
            0, A_PAD, L, unroll=UNROLL,
            carry=jnp.full((L,), -3e38, jnp.float32))(p1)
        row_max = jnp.max(m)

        def p2(off, acc):
            off = pl.multiple_of(off, L)
            v = s_v[j, pl.ds(off, L)]
            p = jnp.exp(v - row_max)
            p = jnp.where(lane + off < A, p, 0.0)
            s_v[j, pl.ds(off, L)] = p
            return acc + p

        acc = plsc.parallel_loop(
            0, A_PAD, L, unroll=UNROLL,
            carry=jnp.zeros((L,), jnp.float32))(p2)
        inv = 1.0 / (jnp.sum(acc) + jnp.zeros((L,), jnp.float32))

        def p3(off):
            off = pl.multiple_of(off, L)
            s_v[j, pl.ds(off, L)] = s_v[j, pl.ds(off, L)] * inv

        plsc.parallel_loop(0, A_PAD, L, unroll=UNROLL)(p3)

    for j in range(NBUF):
        start_in(j, base + j)

    def blk(g, carry):
        for j in range(NBUF):
            i = g * NBUF + j
            row = base + i
            wait_in(j, row)

            @pl.when(g > 0)
            def _():
                pltpu.make_async_copy(
                    s_v.at[j], o_hbm.at[row - NBUF], out_sem.at[j]
                ).wait()

            compute(j, row)
            pltpu.async_copy(s_v.at[j], o_hbm.at[row], out_sem.at[j])

            @pl.when(g < ROWS_PER_W // NBUF - 1)
            def _():
                start_in(j, row + NBUF)
        return carry

    lax.fori_loop(0, ROWS_PER_W // NBUF, blk, 0)

    for j in range(NBUF):
        row = base + ROWS_PER_W - NBUF + j
        pltpu.make_async_copy(
            s_v.at[j], o_hbm.at[row], out_sem.at[j]
        ).wait()


@jax.jit
def _sc_call(action, explore_rate):
    return pl.kernel(
        _sc_body,
        out_type=jax.ShapeDtypeStruct((B, A), jnp.float32),
        mesh=plsc.VectorSubcoreMesh(
            core_axis_name="c", subcore_axis_name="s",
            num_cores=NC, num_subcores=NS,
        ),
        scratch_types=[
            pltpu.VMEM((NBUF, E, A), jnp.float32),
            pltpu.VMEM((NBUF, E, A), jnp.float32),
            pltpu.VMEM((NBUF, A), jnp.float32),
            pltpu.SemaphoreType.DMA((NBUF,)),
            pltpu.SemaphoreType.DMA((NBUF,)),
        ],
        compiler_params=pltpu.CompilerParams(
            use_tc_tiling_on_sc=True, needs_layout_passes=False,
        ),
    )(action, explore_rate)


def kernel(action, explore_rate, step, obs):
    del step, obs
    return _sc_call(action, explore_rate)
